# Initial kernel scaffold; baseline (speedup 1.0000x reference)
import functools
import jax
import jax.numpy as jnp
from jax import lax
from jax.experimental import pallas as pl
from jax.experimental.pallas import tpu as pltpu
from jax.experimental.pallas import tpu_sc as plsc

EMBED = 32
LP = 64
L = 16


def _build(batch, num_cores, num_subcores, interpret=False):
  nw = num_cores * num_subcores
  bpw = batch // nw
  chunk_rows = 2
  chunk_idx = chunk_rows * LP
  nchunk = bpw // chunk_rows
  item_chunks = bpw // 128

  mesh = plsc.VectorSubcoreMesh(
      core_axis_name="c", subcore_axis_name="s",
      num_cores=num_cores, num_subcores=num_subcores)

  @functools.partial(
      pl.kernel,
      out_type=jax.ShapeDtypeStruct((batch, 2 * EMBED), jnp.float32),
      mesh=mesh,
      scratch_types=[
          pltpu.VMEM((nchunk, chunk_idx), jnp.int32),
          pltpu.VMEM((item_chunks, 128), jnp.int32),
          pltpu.VMEM((bpw, EMBED), jnp.float32),
          pltpu.VMEM((chunk_idx, EMBED), jnp.float32),
          pltpu.VMEM((bpw, EMBED), jnp.float32),
          pltpu.SemaphoreType.DMA,
          pltpu.SemaphoreType.DMA,
      ],
      interpret=interpret,
  )
  def sc_kernel(tok_hbm, iid_hbm, itab_hbm, btab_hbm, out_hbm,
                tok_v, iidx_v, irows_v, gbuf_v, pooled_v, isem, gsem):
    wid = lax.axis_index("s") * num_cores + lax.axis_index("c")
    base = wid * bpw

    pltpu.sync_copy(tok_hbm.at[wid], tok_v)
    pltpu.sync_copy(iid_hbm.at[wid], iidx_v)

    for j in range(item_chunks):
      pltpu.async_copy(
          itab_hbm.at[iidx_v.at[j]],
          irows_v.at[pl.ds(j * 128, 128)],
          isem).wait()
    pltpu.sync_copy(irows_v, out_hbm.at[pl.ds(base, bpw), pl.ds(0, EMBED)])

    def chunk_body(j, carry):
      pltpu.async_copy(btab_hbm.at[tok_v.at[j]], gbuf_v, gsem).wait()
      for r in range(chunk_rows):
        acc0 = [jnp.zeros((L,), jnp.float32) for _ in range(4)]
        acc1 = [jnp.zeros((L,), jnp.float32) for _ in range(4)]
        for t in range(LP):
          row = r * LP + t
          acc0[t % 4] += gbuf_v[row, pl.ds(0, L)]
          acc1[t % 4] += gbuf_v[row, pl.ds(L, L)]
        s0 = (acc0[0] + acc0[1]) + (acc0[2] + acc0[3])
        s1 = (acc1[0] + acc1[1]) + (acc1[2] + acc1[3])
        nz = jnp.zeros((L,), jnp.int32)
        for k in range(LP // L):
          t16 = tok_v[j, pl.ds(r * LP + k * L, L)]
          nz += plsc.all_reduce_population_count(t16 != 0)
        inv = 1.0 / jnp.maximum(nz.astype(jnp.float32), 1.0)
        prow = j * chunk_rows + r
        pooled_v[prow, pl.ds(0, L)] = s0 * inv
        pooled_v[prow, pl.ds(L, L)] = s1 * inv
      return carry

    lax.fori_loop(0, nchunk, chunk_body, 0)
    pltpu.sync_copy(pooled_v,
                    out_hbm.at[pl.ds(base, bpw), pl.ds(EMBED, EMBED)])

  return sc_kernel, nw, bpw, nchunk, chunk_idx, item_chunks


def kernel(item_ids, body_tokens, item_table, body_table):
  batch = item_ids.shape[0]
  sc_kernel, nw, bpw, nchunk, chunk_idx, item_chunks = _build(batch, 2, 16)
  iid = item_ids.astype(jnp.int32).reshape(nw, item_chunks, 128)
  tok = jnp.pad(body_tokens.astype(jnp.int32),
                ((0, 0), (0, LP - body_tokens.shape[1])))
  tok = tok.reshape(nw, nchunk, chunk_idx)
  btab = body_table.at[0].set(0.0)
  return sc_kernel(tok, iid, item_table, btab)


# speedup vs baseline: 1.4154x; 1.4154x over previous
import functools
import jax
import jax.numpy as jnp
from jax import lax
from jax.experimental import pallas as pl
from jax.experimental.pallas import tpu as pltpu
from jax.experimental.pallas import tpu_sc as plsc

EMBED = 32
LP = 64
L = 16


def _build(batch, num_cores, num_subcores, interpret=False):
  nw = num_cores * num_subcores
  bpw = batch // nw
  chunk_rows = 2
  chunk_idx = chunk_rows * LP
  nchunk = bpw // chunk_rows
  item_chunks = bpw // 128

  mesh = plsc.VectorSubcoreMesh(
      core_axis_name="c", subcore_axis_name="s",
      num_cores=num_cores, num_subcores=num_subcores)

  @functools.partial(
      pl.kernel,
      out_type=jax.ShapeDtypeStruct((batch, 2 * EMBED), jnp.float32),
      mesh=mesh,
      scratch_types=[
          pltpu.VMEM((nchunk, chunk_idx), jnp.int32),
          pltpu.VMEM((item_chunks, 128), jnp.int32),
          pltpu.VMEM((bpw, EMBED), jnp.float32),
          pltpu.VMEM((chunk_idx, EMBED), jnp.float32),
          pltpu.VMEM((bpw, 2 * EMBED), jnp.float32),
          pltpu.SemaphoreType.DMA,
          pltpu.SemaphoreType.DMA,
      ],
      compiler_params=pltpu.CompilerParams(
          needs_layout_passes=False, use_tc_tiling_on_sc=False),
      interpret=interpret,
  )
  def sc_kernel(tok_hbm, iid_hbm, itab_hbm, btab_hbm, out_hbm,
                tok_v, iidx_v, irows_v, gbuf_v, obuf_v, isem, gsem):
    wid = lax.axis_index("s") * num_cores + lax.axis_index("c")
    base = wid * bpw

    pltpu.sync_copy(tok_hbm.at[wid], tok_v)
    pltpu.sync_copy(iid_hbm.at[wid], iidx_v)

    for j in range(item_chunks):
      pltpu.async_copy(
          itab_hbm.at[iidx_v.at[j]],
          irows_v.at[pl.ds(j * 128, 128)],
          isem).wait()

    def copy_item(r, carry):
      obuf_v[r, pl.ds(0, L)] = irows_v[r, pl.ds(0, L)]
      obuf_v[r, pl.ds(L, L)] = irows_v[r, pl.ds(L, L)]
      return carry
    lax.fori_loop(0, bpw, copy_item, 0)

    def chunk_body(j, carry):
      pltpu.async_copy(btab_hbm.at[tok_v.at[j]], gbuf_v, gsem).wait()
      for r in range(chunk_rows):
        acc0 = [jnp.zeros((L,), jnp.float32) for _ in range(4)]
        acc1 = [jnp.zeros((L,), jnp.float32) for _ in range(4)]
        for t in range(LP):
          row = r * LP + t
          acc0[t % 4] += gbuf_v[row, pl.ds(0, L)]
          acc1[t % 4] += gbuf_v[row, pl.ds(L, L)]
        s0 = (acc0[0] + acc0[1]) + (acc0[2] + acc0[3])
        s1 = (acc1[0] + acc1[1]) + (acc1[2] + acc1[3])
        nz = jnp.zeros((L,), jnp.float32)
        for k in range(LP // L):
          t16 = tok_v[j, pl.ds(r * LP + k * L, L)]
          nz += jnp.where(t16 != 0, 1.0, 0.0)
        cnt = jnp.full((L,), jnp.sum(nz), jnp.float32)
        inv = 1.0 / jnp.maximum(cnt, 1.0)
        prow = j * chunk_rows + r
        obuf_v[prow, pl.ds(2 * L, L)] = s0 * inv
        obuf_v[prow, pl.ds(3 * L, L)] = s1 * inv
      return carry

    lax.fori_loop(0, nchunk, chunk_body, 0)
    pltpu.sync_copy(obuf_v, out_hbm.at[pl.ds(base, bpw)])

  return sc_kernel, nw, bpw, nchunk, chunk_idx, item_chunks


def kernel(item_ids, body_tokens, item_table, body_table):
  batch = item_ids.shape[0]
  sc_kernel, nw, bpw, nchunk, chunk_idx, item_chunks = _build(batch, 2, 16)
  iid = item_ids.astype(jnp.int32).reshape(nw, item_chunks, 128)
  tok = jnp.pad(body_tokens.astype(jnp.int32),
                ((0, 0), (0, LP - body_tokens.shape[1])))
  tok = tok.reshape(nw, nchunk, chunk_idx)
  btab = body_table.at[0].set(0.0)
  return sc_kernel(tok, iid, item_table, btab)


# speedup vs baseline: 1.4176x; 1.0016x over previous
import functools
import jax
import jax.numpy as jnp
from jax import lax
from jax.experimental import pallas as pl
from jax.experimental.pallas import tpu as pltpu
from jax.experimental.pallas import tpu_sc as plsc

EMBED = 32
LP = 64
L = 16
NBUF = 4


def _build(batch, num_cores, num_subcores, interpret=False):
  nw = num_cores * num_subcores
  bpw = batch // nw
  chunk_rows = 2
  chunk_idx = chunk_rows * LP
  nchunk = bpw // chunk_rows
  item_chunks = bpw // 128

  mesh = plsc.VectorSubcoreMesh(
      core_axis_name="c", subcore_axis_name="s",
      num_cores=num_cores, num_subcores=num_subcores)

  @functools.partial(
      pl.kernel,
      out_type=jax.ShapeDtypeStruct((batch, 2 * EMBED), jnp.float32),
      mesh=mesh,
      scratch_types=[
          pltpu.VMEM((nchunk, chunk_idx), jnp.int32),
          pltpu.VMEM((item_chunks, 128), jnp.int32),
          pltpu.VMEM((bpw, EMBED), jnp.float32),
          pltpu.VMEM((NBUF, chunk_idx, EMBED), jnp.float32),
          pltpu.VMEM((bpw, 2 * EMBED), jnp.float32),
          pltpu.SemaphoreType.DMA,
          pltpu.SemaphoreType.DMA((NBUF,)),
      ],
      compiler_params=pltpu.CompilerParams(
          needs_layout_passes=False, use_tc_tiling_on_sc=False),
      interpret=interpret,
  )
  def sc_kernel(tok_hbm, iid_hbm, itab_hbm, btab_hbm, out_hbm,
                tok_v, iidx_v, irows_v, gbuf_v, obuf_v, isem, gsem):
    wid = lax.axis_index("s") * num_cores + lax.axis_index("c")
    base = wid * bpw

    pltpu.sync_copy(tok_hbm.at[wid], tok_v)
    pltpu.sync_copy(iid_hbm.at[wid], iidx_v)

    for j in range(item_chunks):
      pltpu.async_copy(
          itab_hbm.at[iidx_v.at[j]],
          irows_v.at[pl.ds(j * 128, 128)],
          isem)

    for b in range(NBUF):
      pltpu.async_copy(btab_hbm.at[tok_v.at[b]], gbuf_v.at[b], gsem.at[b])

    def compute_chunk(j, b):
      pltpu.make_async_copy(
          btab_hbm.at[pl.ds(0, chunk_idx)], gbuf_v.at[b], gsem.at[b]).wait()
      for r in range(chunk_rows):
        acc0 = [jnp.zeros((L,), jnp.float32) for _ in range(4)]
        acc1 = [jnp.zeros((L,), jnp.float32) for _ in range(4)]
        for t in range(LP):
          row = r * LP + t
          acc0[t % 4] += gbuf_v[b, row, pl.ds(0, L)]
          acc1[t % 4] += gbuf_v[b, row, pl.ds(L, L)]
        s0 = (acc0[0] + acc0[1]) + (acc0[2] + acc0[3])
        s1 = (acc1[0] + acc1[1]) + (acc1[2] + acc1[3])
        nz = jnp.zeros((L,), jnp.float32)
        for k in range(LP // L):
          t16 = tok_v[j, pl.ds(r * LP + k * L, L)]
          nz += jnp.where(t16 != 0, 1.0, 0.0)
        cnt = jnp.full((L,), jnp.sum(nz), jnp.float32)
        inv = 1.0 / jnp.maximum(cnt, 1.0)
        prow = j * chunk_rows + r
        obuf_v[prow, pl.ds(2 * L, L)] = s0 * inv
        obuf_v[prow, pl.ds(3 * L, L)] = s1 * inv
      @pl.when(j + NBUF < nchunk)
      def _():
        pltpu.async_copy(
            btab_hbm.at[tok_v.at[j + NBUF]], gbuf_v.at[b], gsem.at[b])

    @pl.loop(0, nchunk, step=NBUF)
    def _(g):
      for b in range(NBUF):
        compute_chunk(g + b, b)

    pltpu.make_async_copy(
        itab_hbm.at[pl.ds(0, bpw)], irows_v, isem).wait()

    def copy_item(r, carry):
      obuf_v[r, pl.ds(0, L)] = irows_v[r, pl.ds(0, L)]
      obuf_v[r, pl.ds(L, L)] = irows_v[r, pl.ds(L, L)]
      return carry
    lax.fori_loop(0, bpw, copy_item, 0)

    pltpu.sync_copy(obuf_v, out_hbm.at[pl.ds(base, bpw)])

  return sc_kernel, nw, bpw, nchunk, chunk_idx, item_chunks


def kernel(item_ids, body_tokens, item_table, body_table):
  batch = item_ids.shape[0]
  sc_kernel, nw, bpw, nchunk, chunk_idx, item_chunks = _build(batch, 2, 16)
  iid = item_ids.astype(jnp.int32).reshape(nw, item_chunks, 128)
  tok = jnp.pad(body_tokens.astype(jnp.int32),
                ((0, 0), (0, LP - body_tokens.shape[1])))
  tok = tok.reshape(nw, nchunk, chunk_idx)
  btab = body_table.at[0].set(0.0)
  return sc_kernel(tok, iid, item_table, btab)
